# initial kernel scaffold (unmeasured)
import jax
import jax.numpy as jnp
from jax import lax
from jax.experimental import pallas as pl
from jax.experimental.pallas import tpu as pltpu

N_DEV = 8
B, H, D, BS = 8, 8, 64, 16
NEG = -1e30


def kernel(Q, K, V, bt, lens):
    n_pages = K.shape[0]
    n_keys = n_pages * K.shape[1]
    nbt = bt.shape[1]

    def body(q_ref, k_ref, v_ref, bt_ref, lens_ref, out_ref,
             o_comm, m_comm, l_comm,
             o_send, o_recv, m_send, m_recv, l_send, l_recv):
        my = lax.axis_index("i")

        q = jnp.reshape(q_ref[...], (B, H, D))
        k = jnp.reshape(k_ref[...], (n_keys, H, D))
        v = jnp.reshape(v_ref[...], (n_keys, H, D))
        btv = bt_ref[...]
        lensv = lens_ref[...]

        j_iota = lax.broadcasted_iota(jnp.int32, (B, nbt), 1)
        valid = j_iota < lensv
        base = my * n_pages
        pg = lax.broadcasted_iota(jnp.int32, (B, nbt, n_pages), 2)
        match = (btv[:, :, None] - base == pg) & valid[:, :, None]
        w = jnp.sum(match.astype(jnp.float32), axis=1)
        w_key = jnp.reshape(
            jnp.broadcast_to(w[:, :, None], (B, n_pages, BS)), (B, n_keys))

        s = lax.dot_general(q, k, (((2,), (2,)), ((1,), (1,))),
                            preferred_element_type=jnp.float32)
        s = s * (D ** -0.5)
        s = jnp.where((w_key > 0.0)[None, :, :], s, NEG)
        m = jnp.max(s, axis=-1)
        p = w_key[None] * jnp.exp(s - m[:, :, None])
        lsum = jnp.sum(p, axis=-1)
        o = lax.dot_general(p, v, (((2,), (0,)), ((0,), (1,))),
                            preferred_element_type=jnp.float32)

        o_comm[0] = o
        m_comm[0] = m
        l_comm[0] = lsum

        bsem = pltpu.get_barrier_semaphore()
        for d in range(1, N_DEV):
            peer = (my + d) % N_DEV
            pl.semaphore_signal(bsem, inc=1, device_id=(peer,),
                                device_id_type=pl.DeviceIdType.MESH)
        pl.semaphore_wait(bsem, N_DEV - 1)

        descs = []
        for d in range(1, N_DEV):
            peer = (my + d) % N_DEV
            for buf, ssem, rsem in ((o_comm, o_send, o_recv),
                                    (m_comm, m_send, m_recv),
                                    (l_comm, l_send, l_recv)):
                rdma = pltpu.make_async_remote_copy(
                    src_ref=buf.at[0], dst_ref=buf.at[d],
                    send_sem=ssem.at[d], recv_sem=rsem.at[d],
                    device_id=(peer,), device_id_type=pl.DeviceIdType.MESH)
                rdma.start()
                descs.append(rdma)

        for rdma in descs:
            rdma.wait_recv()

        o_all = o_comm[...]
        m_all = m_comm[...]
        l_all = l_comm[...]
        mg = jnp.max(m_all, axis=0)
        alpha = jnp.exp(m_all - mg[None])
        lg = jnp.sum(l_all * alpha, axis=0)
        og = jnp.sum(o_all * alpha[..., None], axis=0)
        res = og / lg[..., None]
        out_ref[...] = jnp.reshape(jnp.transpose(res, (1, 0, 2)),
                                   (B, 1, H, D))

        for rdma in descs:
            rdma.wait_send()

    return pl.pallas_call(
        body,
        out_shape=jax.ShapeDtypeStruct((B, 1, H, D), jnp.float32),
        in_specs=[pl.BlockSpec(memory_space=pltpu.VMEM)] * 5,
        out_specs=pl.BlockSpec(memory_space=pltpu.VMEM),
        scratch_shapes=[
            pltpu.VMEM((N_DEV, H, B, D), jnp.float32),
            pltpu.VMEM((N_DEV, H, B), jnp.float32),
            pltpu.VMEM((N_DEV, H, B), jnp.float32),
            pltpu.SemaphoreType.DMA((N_DEV,)),
            pltpu.SemaphoreType.DMA((N_DEV,)),
            pltpu.SemaphoreType.DMA((N_DEV,)),
            pltpu.SemaphoreType.DMA((N_DEV,)),
            pltpu.SemaphoreType.DMA((N_DEV,)),
            pltpu.SemaphoreType.DMA((N_DEV,)),
        ],
        compiler_params=pltpu.CompilerParams(collective_id=0),
    )(Q, K, V, bt, lens.reshape(B, 1))


# baseline (device time: 35146 ns/iter reference)
import jax
import jax.numpy as jnp
from jax import lax
from jax.experimental import pallas as pl
from jax.experimental.pallas import tpu as pltpu

N_DEV = 8
B, H, D, BS = 8, 8, 64, 16
NEG = -1e30


def kernel(Q, K, V, bt, lens):
    n_pages = K.shape[0]
    n_keys = n_pages * K.shape[1]
    nbt = bt.shape[1]

    def body(q_ref, k_ref, v_ref, bt_ref, lens_ref, out_ref,
             o_comm, m_comm, l_comm,
             o_send, o_recv, m_send, m_recv, l_send, l_recv):
        my = lax.axis_index("i")

        q = jnp.reshape(q_ref[...], (B, H, D))
        k = jnp.reshape(k_ref[...], (n_keys, H, D))
        v = jnp.reshape(v_ref[...], (n_keys, H, D))
        btv = bt_ref[...]
        lensv = lens_ref[...]

        j_iota = lax.broadcasted_iota(jnp.int32, (B, nbt), 1)
        base = my * n_pages
        btv_m = jnp.where(j_iota < lensv, btv - base, -1)
        pk = lax.broadcasted_iota(jnp.int32, (B, nbt, n_keys), 2) // BS
        match = btv_m[:, :, None] == pk
        w_key = jnp.sum(match.astype(jnp.float32), axis=1)
        logw = jnp.where(w_key > 0.0,
                         jnp.log(jnp.maximum(w_key, 1e-30)), NEG)

        s = lax.dot_general(q, k, (((2,), (2,)), ((1,), (1,))),
                            preferred_element_type=jnp.float32)
        s = s * (D ** -0.5) + logw[None]
        m = jnp.max(s, axis=-1)
        p = jnp.exp(s - m[:, :, None])
        lsum = jnp.sum(p, axis=-1)
        o = lax.dot_general(p, v, (((2,), (0,)), ((0,), (1,))),
                            preferred_element_type=jnp.float32)

        o_comm[0] = o
        m_comm[0] = m
        l_comm[0] = lsum

        bsem = pltpu.get_barrier_semaphore()
        for d in range(1, N_DEV):
            peer = (my + d) % N_DEV
            pl.semaphore_signal(bsem, inc=1, device_id=(peer,),
                                device_id_type=pl.DeviceIdType.MESH)
        pl.semaphore_wait(bsem, N_DEV - 1)

        descs = []
        for d in range(1, N_DEV):
            peer = (my + d) % N_DEV
            for buf, ssem, rsem in ((o_comm, o_send, o_recv),
                                    (m_comm, m_send, m_recv),
                                    (l_comm, l_send, l_recv)):
                rdma = pltpu.make_async_remote_copy(
                    src_ref=buf.at[0], dst_ref=buf.at[d],
                    send_sem=ssem.at[d], recv_sem=rsem.at[d],
                    device_id=(peer,), device_id_type=pl.DeviceIdType.MESH)
                rdma.start()
                descs.append(rdma)

        for rdma in descs:
            rdma.wait_recv()

        o_all = o_comm[...]
        m_all = m_comm[...]
        l_all = l_comm[...]
        mg = jnp.max(m_all, axis=0)
        alpha = jnp.exp(m_all - mg[None])
        lg = jnp.sum(l_all * alpha, axis=0)
        og = jnp.sum(o_all * alpha[..., None], axis=0)
        res = og / lg[..., None]
        out_ref[...] = jnp.reshape(jnp.transpose(res, (1, 0, 2)),
                                   (B, 1, H, D))

        for rdma in descs:
            rdma.wait_send()

    return pl.pallas_call(
        body,
        out_shape=jax.ShapeDtypeStruct((B, 1, H, D), jnp.float32),
        in_specs=[pl.BlockSpec(memory_space=pltpu.VMEM)] * 5,
        out_specs=pl.BlockSpec(memory_space=pltpu.VMEM),
        scratch_shapes=[
            pltpu.VMEM((N_DEV, H, B, D), jnp.float32),
            pltpu.VMEM((N_DEV, H, B), jnp.float32),
            pltpu.VMEM((N_DEV, H, B), jnp.float32),
            pltpu.SemaphoreType.DMA((N_DEV,)),
            pltpu.SemaphoreType.DMA((N_DEV,)),
            pltpu.SemaphoreType.DMA((N_DEV,)),
            pltpu.SemaphoreType.DMA((N_DEV,)),
            pltpu.SemaphoreType.DMA((N_DEV,)),
            pltpu.SemaphoreType.DMA((N_DEV,)),
        ],
        compiler_params=pltpu.CompilerParams(
            collective_id=0, vmem_limit_bytes=100 * 1024 * 1024),
    )(Q, K, V, bt, lens.reshape(B, 1))


# device time: 26587 ns/iter; 1.3219x vs baseline; 1.3219x over previous
import jax
import jax.numpy as jnp
from jax import lax
from jax.experimental import pallas as pl
from jax.experimental.pallas import tpu as pltpu

N_DEV = 8
B, H, D, BS = 8, 8, 64, 16
R = B * H
NEG = -1e30


def kernel(Q, K, V, bt, lens):
    n_pages = K.shape[0]
    n_keys = n_pages * K.shape[1]
    nbt = bt.shape[1]

    def body(q_ref, k_ref, v_ref, bt_ref, lens_ref, out_ref,
             o_comm, m_comm, l_comm,
             o_send, o_recv, m_send, m_recv, l_send, l_recv):
        my = lax.axis_index("i")

        qf = q_ref[...]
        btv = bt_ref[...]
        lensv = lens_ref[...]

        j_iota = lax.broadcasted_iota(jnp.int32, (B, nbt), 1)
        base = my * n_pages
        btv_m = jnp.where(j_iota < lensv, btv - base, -1)
        pk = lax.broadcasted_iota(jnp.int32, (B, nbt, n_keys), 2) // BS
        match = btv_m[:, :, None] == pk
        w_key = jnp.sum(match.astype(jnp.float32), axis=1)
        logw = jnp.where(w_key > 0.0,
                         jnp.log(jnp.maximum(w_key, 1e-30)), NEG)

        rep = (lax.broadcasted_iota(jnp.int32, (R, B), 0) // H
               == lax.broadcasted_iota(jnp.int32, (R, B), 1)
               ).astype(jnp.float32)
        hmask = (lax.broadcasted_iota(jnp.int32, (R, H * D), 0) % H
                 == lax.broadcasted_iota(jnp.int32, (R, H * D), 1) // D
                 ).astype(jnp.float32)
        qbig = lax.dot_general(rep, qf, (((1,), (0,)), ((), ())),
                               preferred_element_type=jnp.float32) * hmask
        logw_r = lax.dot_general(rep, logw, (((1,), (0,)), ((), ())),
                                 preferred_element_type=jnp.float32)

        s = lax.dot_general(qbig, k_ref[...], (((1,), (1,)), ((), ())),
                            preferred_element_type=jnp.float32)
        s = s * (D ** -0.5) + logw_r
        m = jnp.max(s, axis=-1, keepdims=True)
        p = jnp.exp(s - m)
        lsum = jnp.sum(p, axis=-1, keepdims=True)
        o2 = lax.dot_general(p, v_ref[...], (((1,), (0,)), ((), ())),
                             preferred_element_type=jnp.float32) * hmask
        o = o2[:, 0:D]
        for blk in range(1, H):
            o = o + o2[:, blk * D:(blk + 1) * D]

        o_comm[0] = o
        m_comm[0] = m
        l_comm[0] = lsum

        bsem = pltpu.get_barrier_semaphore()
        for d in range(1, N_DEV):
            peer = (my + d) % N_DEV
            pl.semaphore_signal(bsem, inc=1, device_id=(peer,),
                                device_id_type=pl.DeviceIdType.MESH)
        pl.semaphore_wait(bsem, N_DEV - 1)

        descs = []
        for d in range(1, N_DEV):
            peer = (my + d) % N_DEV
            for buf, ssem, rsem in ((o_comm, o_send, o_recv),
                                    (m_comm, m_send, m_recv),
                                    (l_comm, l_send, l_recv)):
                rdma = pltpu.make_async_remote_copy(
                    src_ref=buf.at[0], dst_ref=buf.at[d],
                    send_sem=ssem.at[d], recv_sem=rsem.at[d],
                    device_id=(peer,), device_id_type=pl.DeviceIdType.MESH)
                rdma.start()
                descs.append(rdma)

        for rdma in descs:
            rdma.wait_recv()

        o_all = o_comm[...]
        m_all = m_comm[...]
        l_all = l_comm[...]
        mg = jnp.max(m_all, axis=0)
        alpha = jnp.exp(m_all - mg[None])
        lg = jnp.sum(l_all * alpha, axis=0)
        og = jnp.sum(o_all * alpha, axis=0)
        out_ref[...] = jnp.reshape(og / lg, (B, 1, H, D))

        for rdma in descs:
            rdma.wait_send()

    return pl.pallas_call(
        body,
        out_shape=jax.ShapeDtypeStruct((B, 1, H, D), jnp.float32),
        in_specs=[pl.BlockSpec(memory_space=pltpu.VMEM)] * 5,
        out_specs=pl.BlockSpec(memory_space=pltpu.VMEM),
        scratch_shapes=[
            pltpu.VMEM((N_DEV, R, D), jnp.float32),
            pltpu.VMEM((N_DEV, R, 1), jnp.float32),
            pltpu.VMEM((N_DEV, R, 1), jnp.float32),
            pltpu.SemaphoreType.DMA((N_DEV,)),
            pltpu.SemaphoreType.DMA((N_DEV,)),
            pltpu.SemaphoreType.DMA((N_DEV,)),
            pltpu.SemaphoreType.DMA((N_DEV,)),
            pltpu.SemaphoreType.DMA((N_DEV,)),
            pltpu.SemaphoreType.DMA((N_DEV,)),
        ],
        compiler_params=pltpu.CompilerParams(
            collective_id=0, vmem_limit_bytes=100 * 1024 * 1024),
    )(Q.reshape(B, H * D), K.reshape(n_keys, H * D), V.reshape(n_keys, H * D),
      bt, lens.reshape(B, 1))


# device time: 16513 ns/iter; 2.1284x vs baseline; 1.6101x over previous
import jax
import jax.numpy as jnp
from jax import lax
from jax.experimental import pallas as pl
from jax.experimental.pallas import tpu as pltpu

N_DEV = 8
B, H, D, BS = 8, 8, 64, 16
R = B * H
W = D + 2
NEG = -1e30


def kernel(Q, K, V, bt, lens):
    n_pages = K.shape[0]
    n_keys = n_pages * K.shape[1]
    nbt = bt.shape[1]

    def body(q_ref, k_ref, v_ref, bt_ref, lens_ref, out_ref,
             stats, send_sems, recv_sems):
        my = lax.axis_index("i")

        bsem = pltpu.get_barrier_semaphore()
        for d in range(1, N_DEV):
            peer = (my + d) % N_DEV
            pl.semaphore_signal(bsem, inc=1, device_id=(peer,),
                                device_id_type=pl.DeviceIdType.MESH)

        qf = jnp.reshape(q_ref[...], (B, H * D))
        btv = bt_ref[...]
        lensv = lens_ref[...]

        j_iota = lax.broadcasted_iota(jnp.int32, (B, nbt), 1)
        base = my * n_pages
        btv_m = jnp.where(j_iota < lensv, btv - base, -1)
        pk = lax.broadcasted_iota(jnp.int32, (B, nbt, n_keys), 2) // BS
        match = btv_m[:, :, None] == pk
        w_key = jnp.sum(match.astype(jnp.float32), axis=1)
        logw = jnp.where(w_key > 0.0,
                         jnp.log(jnp.maximum(w_key, 1e-30)), NEG)

        rep = (lax.broadcasted_iota(jnp.int32, (R, B), 0) // H
               == lax.broadcasted_iota(jnp.int32, (R, B), 1)
               ).astype(jnp.float32)
        hmask = (lax.broadcasted_iota(jnp.int32, (R, H * D), 0) % H
                 == lax.broadcasted_iota(jnp.int32, (R, H * D), 1) // D
                 ).astype(jnp.float32)
        qbig = lax.dot_general(rep, qf, (((1,), (0,)), ((), ())),
                               preferred_element_type=jnp.float32) * hmask
        logw_r = lax.dot_general(rep, logw, (((1,), (0,)), ((), ())),
                                 preferred_element_type=jnp.float32)

        k2 = jnp.reshape(k_ref[...], (n_keys, H * D))
        s = lax.dot_general(qbig, k2, (((1,), (1,)), ((), ())),
                            preferred_element_type=jnp.float32)
        s = s * (D ** -0.5) + logw_r
        m = jnp.max(s, axis=-1, keepdims=True)
        p = jnp.exp(s - m)
        lsum = jnp.sum(p, axis=-1, keepdims=True)
        v2 = jnp.reshape(v_ref[...], (n_keys, H * D))
        o2 = lax.dot_general(p, v2, (((1,), (0,)), ((), ())),
                             preferred_element_type=jnp.float32) * hmask
        o = o2[:, 0:D]
        for blk in range(1, H):
            o = o + o2[:, blk * D:(blk + 1) * D]

        stats[0] = jnp.concatenate([o, m, lsum], axis=1)

        pl.semaphore_wait(bsem, N_DEV - 1)
        descs = []
        for d in range(1, N_DEV):
            peer = (my + d) % N_DEV
            rdma = pltpu.make_async_remote_copy(
                src_ref=stats.at[0], dst_ref=stats.at[d],
                send_sem=send_sems.at[d], recv_sem=recv_sems.at[d],
                device_id=(peer,), device_id_type=pl.DeviceIdType.MESH)
            rdma.start()
            descs.append(rdma)

        for rdma in descs:
            rdma.wait_recv()

        st = stats[...]
        o_all = st[:, :, 0:D]
        m_all = st[:, :, D:D + 1]
        l_all = st[:, :, D + 1:W]
        mg = jnp.max(m_all, axis=0)
        alpha = jnp.exp(m_all - mg[None])
        lg = jnp.sum(l_all * alpha, axis=0)
        og = jnp.sum(o_all * alpha, axis=0)
        out_ref[...] = jnp.reshape(og / lg, (B, 1, H, D))

        for rdma in descs:
            rdma.wait_send()

    return pl.pallas_call(
        body,
        out_shape=jax.ShapeDtypeStruct((B, 1, H, D), jnp.float32),
        in_specs=[pl.BlockSpec(memory_space=pltpu.VMEM)] * 5,
        out_specs=pl.BlockSpec(memory_space=pltpu.VMEM),
        scratch_shapes=[
            pltpu.VMEM((N_DEV, R, W), jnp.float32),
            pltpu.SemaphoreType.DMA((N_DEV,)),
            pltpu.SemaphoreType.DMA((N_DEV,)),
        ],
        compiler_params=pltpu.CompilerParams(
            collective_id=0, vmem_limit_bytes=100 * 1024 * 1024),
    )(Q, K, V, bt, lens.reshape(B, 1))
